# baseline (speedup 1.0000x reference)
import jax
import jax.numpy as jnp
from jax.experimental import pallas as pl
from jax.experimental.pallas import tpu as pltpu

MAX_H_BOUND = 1080.0
MAX_W_BOUND = 1920.0


def _chamfer_step_kernel(c_ref, v_ref, tgt_ref, o_ref):
    c = c_ref[0, 0]
    v = v_ref[0, 0]
    tgt = tgt_ref[0, 0]

    px = jnp.clip(c[:, 0:1] + v[:, 0:1], 0.0, MAX_H_BOUND)
    py = jnp.clip(c[:, 1:2] + v[:, 1:2], 0.0, MAX_W_BOUND)
    qx = tgt[0:1, :]
    qy = tgt[1:2, :]

    pxb = px.astype(jnp.bfloat16).astype(jnp.float32)
    pyb = py.astype(jnp.bfloat16).astype(jnp.float32)
    qxb = qx.astype(jnp.bfloat16).astype(jnp.float32)
    qyb = qy.astype(jnp.bfloat16).astype(jnp.float32)

    p2 = px * px + py * py
    q2 = qx * qx + qy * qy
    pq = pxb * qxb + pyb * qyb
    d2 = (p2 + q2) - 2.0 * pq

    row_min = jnp.min(d2, axis=1, keepdims=True)
    col_min = jnp.min(d2, axis=0, keepdims=True)
    partial = jnp.sum(jnp.sqrt(jnp.maximum(row_min, 0.0))) + jnp.sum(
        jnp.sqrt(jnp.maximum(col_min, 0.0))
    )

    o_ref[...] = jnp.full(o_ref.shape, partial, dtype=jnp.float32)


def kernel(center_tensor, velocity_vector):
    N, T, C, _ = center_tensor.shape
    Tm = T - 1
    center_t = center_tensor.transpose(0, 1, 3, 2)

    out = pl.pallas_call(
        _chamfer_step_kernel,
        grid=(N, Tm),
        in_specs=[
            pl.BlockSpec((1, 1, C, 2), lambda n, t: (n, t, 0, 0)),
            pl.BlockSpec((1, 1, C, 2), lambda n, t: (n, t, 0, 0)),
            pl.BlockSpec((1, 1, 2, C), lambda n, t: (n, t + 1, 0, 0)),
        ],
        out_specs=pl.BlockSpec((1, 1, 8, 128), lambda n, t: (n, t, 0, 0)),
        out_shape=jax.ShapeDtypeStruct((N, Tm, 8, 128), jnp.float32),
        compiler_params=pltpu.CompilerParams(
            dimension_semantics=("parallel", "arbitrary"),
        ),
        name="chamfer_step",
    )(center_tensor, velocity_vector, center_t)

    total = jnp.sum(out[:, :, 0, 0])
    return total * (0.5 / (N * C * Tm))


# speedup vs baseline: 1.1856x; 1.1856x over previous
import jax
import jax.numpy as jnp
from jax.experimental import pallas as pl
from jax.experimental.pallas import tpu as pltpu

MAX_H_BOUND = 1080.0
MAX_W_BOUND = 1920.0

TB = 4
RB = 128


def _bf16_round(x):
    return x.astype(jnp.bfloat16).astype(jnp.float32)


def _chamfer_steps_kernel(c_ref, v_ref, tgt_ref, o_ref):
    C = c_ref.shape[2]
    racc = jnp.zeros((RB, 1), jnp.float32)
    cacc = jnp.zeros((1, C), jnp.float32)
    for k in range(TB):
        c = c_ref[0, k]
        v = v_ref[0, k]
        tgt = tgt_ref[0, k]

        s = c + v
        sx = jnp.clip(s[:, 0:1], 0.0, MAX_H_BOUND)
        sy = jnp.clip(s[:, 1:2], 0.0, MAX_W_BOUND)
        qx = tgt[0:1, :]
        qy = tgt[1:2, :]

        pxb = _bf16_round(sx)
        pyb = _bf16_round(sy)
        qxb = _bf16_round(qx)
        qyb = _bf16_round(qy)

        p2 = sx * sx + sy * sy
        q2 = qx * qx + qy * qy

        col_min = None
        for rb in range(C // RB):
            sl = slice(rb * RB, (rb + 1) * RB)
            pq = pxb[sl] * qxb + pyb[sl] * qyb
            d2 = (p2[sl] + q2) - 2.0 * pq
            rmin = jnp.min(d2, axis=1, keepdims=True)
            racc = racc + jnp.sqrt(jnp.maximum(rmin, 0.0))
            cmin = jnp.min(d2, axis=0, keepdims=True)
            col_min = cmin if col_min is None else jnp.minimum(col_min, cmin)
        cacc = cacc + jnp.sqrt(jnp.maximum(col_min, 0.0))

    partial = jnp.sum(racc) + jnp.sum(cacc)
    o_ref[...] = jnp.full(o_ref.shape, partial, dtype=jnp.float32)


def kernel(center_tensor, velocity_vector):
    N, T, C, _ = center_tensor.shape
    Tm = T - 1
    tgt_t = center_tensor[:, 1:].transpose(0, 1, 3, 2)

    out = pl.pallas_call(
        _chamfer_steps_kernel,
        grid=(N, Tm // TB),
        in_specs=[
            pl.BlockSpec((1, TB, C, 2), lambda n, t: (n, t, 0, 0)),
            pl.BlockSpec((1, TB, C, 2), lambda n, t: (n, t, 0, 0)),
            pl.BlockSpec((1, TB, 2, C), lambda n, t: (n, t, 0, 0)),
        ],
        out_specs=pl.BlockSpec((1, 1, 8, 128), lambda n, t: (n, t, 0, 0)),
        out_shape=jax.ShapeDtypeStruct((N, Tm // TB, 8, 128), jnp.float32),
        compiler_params=pltpu.CompilerParams(
            dimension_semantics=("parallel", "arbitrary"),
        ),
        name="chamfer_steps",
    )(center_tensor[:, :-1], velocity_vector[:, :-1], tgt_t)

    total = jnp.sum(out[:, :, 0, 0])
    return total * (0.5 / (N * C * Tm))


# speedup vs baseline: 2.8318x; 2.3885x over previous
import jax
import jax.numpy as jnp
from jax.experimental import pallas as pl
from jax.experimental.pallas import tpu as pltpu

MAX_H_BOUND = 1080.0
MAX_W_BOUND = 1920.0

TB = 4

_DN = (((0,), (0,)), ((), ()))


def _lane_min(x):
    n = x.shape[1]
    while n > 128:
        n //= 2
        x = jnp.minimum(x[:, :n], x[:, n:])
    return jnp.min(x, axis=1, keepdims=True)


def _sublane_min(x):
    n = x.shape[0]
    while n > 1:
        n //= 2
        x = jnp.minimum(x[:n, :], x[n:, :])
    return x


def _chamfer_steps_kernel(p_ref, v_ref, q_ref, o_ref):
    C = p_ref.shape[3]
    racc = jnp.zeros((C, 1), jnp.float32)
    cacc = jnp.zeros((1, C), jnp.float32)
    for k in range(TB):
        pc = p_ref[0, k]
        vv = v_ref[0, k]
        qc = q_ref[0, k]

        s = pc + vv
        sx = jnp.clip(s[0:1, :], 0.0, MAX_H_BOUND)
        sy = jnp.clip(s[1:2, :], 0.0, MAX_W_BOUND)
        qx = qc[0:1, :]
        qy = qc[1:2, :]

        p2 = sx * sx + sy * sy
        q2 = qx * qx + qy * qy
        p2c = jnp.transpose(p2)

        pb2 = (-2.0 * jnp.concatenate([sx, sy], axis=0)).astype(jnp.bfloat16)
        qb = qc.astype(jnp.bfloat16)

        m = jax.lax.dot_general(pb2, qb, _DN, preferred_element_type=jnp.float32)

        rmin = _lane_min(q2 + m)
        cmin = _sublane_min(p2c + m)

        racc = racc + jnp.sqrt(jnp.maximum(rmin + p2c, 0.0))
        cacc = cacc + jnp.sqrt(jnp.maximum(cmin + q2, 0.0))

    partial = jnp.sum(racc) + jnp.sum(cacc)
    o_ref[...] = jnp.full(o_ref.shape, partial, dtype=jnp.float32)


def kernel(center_tensor, velocity_vector):
    N, T, C, _ = center_tensor.shape
    Tm = T - 1
    ct = center_tensor.transpose(0, 1, 3, 2)
    vt = velocity_vector.transpose(0, 1, 3, 2)

    out = pl.pallas_call(
        _chamfer_steps_kernel,
        grid=(N, Tm // TB),
        in_specs=[
            pl.BlockSpec((1, TB, 2, C), lambda n, t: (n, t, 0, 0)),
            pl.BlockSpec((1, TB, 2, C), lambda n, t: (n, t, 0, 0)),
            pl.BlockSpec((1, TB, 2, C), lambda n, t: (n, t, 0, 0)),
        ],
        out_specs=pl.BlockSpec((1, 1, 8, 128), lambda n, t: (n, t, 0, 0)),
        out_shape=jax.ShapeDtypeStruct((N, Tm // TB, 8, 128), jnp.float32),
        compiler_params=pltpu.CompilerParams(
            dimension_semantics=("parallel", "arbitrary"),
        ),
        name="chamfer_steps",
    )(ct[:, :-1], vt[:, :-1], ct[:, 1:])

    total = jnp.sum(out[:, :, 0, 0])
    return total * (0.5 / (N * C * Tm))


# speedup vs baseline: 3.3884x; 1.1966x over previous
import jax
import jax.numpy as jnp
from jax.experimental import pallas as pl
from jax.experimental.pallas import tpu as pltpu

MAX_H_BOUND = 1080.0
MAX_W_BOUND = 1920.0

TB = 16

_DN = (((0,), (0,)), ((), ()))


def _lane_min(x):
    n = x.shape[1]
    while n > 128:
        n //= 2
        x = jnp.minimum(x[:, :n], x[:, n:])
    return jnp.min(x, axis=1, keepdims=True)


def _sublane_min(x):
    n = x.shape[0]
    while n > 1:
        n //= 2
        x = jnp.minimum(x[:n, :], x[n:, :])
    return x


def _chamfer_steps_kernel(p_ref, v_ref, q_ref, o_ref):
    C = p_ref.shape[3]
    racc = jnp.zeros((C, 1), jnp.float32)
    cacc = jnp.zeros((1, C), jnp.float32)
    for k in range(TB):
        pc = p_ref[0, k]
        vv = v_ref[0, k]
        qc = q_ref[0, k]

        s = pc + vv
        sx = jnp.clip(s[0:1, :], 0.0, MAX_H_BOUND)
        sy = jnp.clip(s[1:2, :], 0.0, MAX_W_BOUND)
        qx = qc[0:1, :]
        qy = qc[1:2, :]

        p2 = sx * sx + sy * sy
        q2 = qx * qx + qy * qy
        p2c = jnp.transpose(p2)

        pb2 = (-2.0 * jnp.concatenate([sx, sy], axis=0)).astype(jnp.bfloat16)
        qb = qc.astype(jnp.bfloat16)

        m = jax.lax.dot_general(pb2, qb, _DN, preferred_element_type=jnp.float32)

        rmin = _lane_min(q2 + m)
        cmin = _sublane_min(p2c + m)

        racc = racc + jnp.sqrt(jnp.maximum(rmin + p2c, 0.0))
        cacc = cacc + jnp.sqrt(jnp.maximum(cmin + q2, 0.0))

    partial = jnp.sum(racc) + jnp.sum(cacc)
    o_ref[...] = jnp.full(o_ref.shape, partial, dtype=jnp.float32)


def kernel(center_tensor, velocity_vector):
    N, T, C, _ = center_tensor.shape
    Tm = T - 1
    ct = center_tensor.transpose(0, 1, 3, 2)
    vt = velocity_vector.transpose(0, 1, 3, 2)

    out = pl.pallas_call(
        _chamfer_steps_kernel,
        grid=(N, Tm // TB),
        in_specs=[
            pl.BlockSpec((1, TB, 2, C), lambda n, t: (n, t, 0, 0)),
            pl.BlockSpec((1, TB, 2, C), lambda n, t: (n, t, 0, 0)),
            pl.BlockSpec((1, TB, 2, C), lambda n, t: (n, t, 0, 0)),
        ],
        out_specs=pl.BlockSpec((1, 1, 8, 128), lambda n, t: (n, t, 0, 0)),
        out_shape=jax.ShapeDtypeStruct((N, Tm // TB, 8, 128), jnp.float32),
        compiler_params=pltpu.CompilerParams(
            dimension_semantics=("parallel", "arbitrary"),
        ),
        name="chamfer_steps",
    )(ct[:, :-1], vt[:, :-1], ct[:, 1:])

    total = jnp.sum(out[:, :, 0, 0])
    return total * (0.5 / (N * C * Tm))


# speedup vs baseline: 5.0307x; 1.4847x over previous
import jax
import jax.numpy as jnp
from jax.experimental import pallas as pl
from jax.experimental.pallas import tpu as pltpu

MAX_H_BOUND = 1080.0
MAX_W_BOUND = 1920.0

TB = 16

_DN = (((0,), (0,)), ((), ()))


def _bf16x3(x):
    hi = x.astype(jnp.bfloat16)
    r = x - hi.astype(jnp.float32)
    mid = r.astype(jnp.bfloat16)
    lo = (r - mid.astype(jnp.float32)).astype(jnp.bfloat16)
    return hi, mid, lo


def _lane_min(x):
    n = x.shape[1]
    while n > 128:
        n //= 2
        x = jnp.minimum(x[:, :n], x[:, n:])
    return jnp.min(x, axis=1, keepdims=True)


def _sublane_min(x):
    n = x.shape[0]
    while n > 1:
        n //= 2
        x = jnp.minimum(x[:n, :], x[n:, :])
    return x


def _chamfer_steps_kernel(p_ref, v_ref, q_ref, o_ref):
    C = p_ref.shape[3]
    racc = jnp.zeros((C, 1), jnp.float32)
    cacc = jnp.zeros((1, C), jnp.float32)
    for k in range(TB):
        pc = p_ref[0, k]
        vv = v_ref[0, k]
        qc = q_ref[0, k]

        s = pc + vv
        sx = jnp.clip(s[0:1, :], 0.0, MAX_H_BOUND)
        sy = jnp.clip(s[1:2, :], 0.0, MAX_W_BOUND)
        qx = qc[0:1, :]
        qy = qc[1:2, :]

        p2 = sx * sx + sy * sy
        q2 = qx * qx + qy * qy

        ones = jnp.ones((1, C), jnp.bfloat16)
        p2h, p2m, p2l = _bf16x3(p2)
        q2h, q2m, q2l = _bf16x3(q2)

        lhs = jnp.concatenate(
            [
                (-2.0 * sx).astype(jnp.bfloat16),
                (-2.0 * sy).astype(jnp.bfloat16),
                p2h, p2m, p2l,
                ones, ones, ones,
            ],
            axis=0,
        )
        rhs = jnp.concatenate(
            [
                qc.astype(jnp.bfloat16),
                ones, ones, ones,
                q2h, q2m, q2l,
            ],
            axis=0,
        )
        d2 = jax.lax.dot_general(lhs, rhs, _DN, preferred_element_type=jnp.float32)

        rmin = _lane_min(d2)
        cmin = _sublane_min(d2)

        racc = racc + jnp.sqrt(jnp.maximum(rmin, 0.0))
        cacc = cacc + jnp.sqrt(jnp.maximum(cmin, 0.0))

    partial = jnp.sum(racc) + jnp.sum(cacc)
    o_ref[...] = jnp.full(o_ref.shape, partial, dtype=jnp.float32)


def kernel(center_tensor, velocity_vector):
    N, T, C, _ = center_tensor.shape
    Tm = T - 1
    ct = center_tensor.transpose(0, 1, 3, 2)
    vt = velocity_vector.transpose(0, 1, 3, 2)

    out = pl.pallas_call(
        _chamfer_steps_kernel,
        grid=(N, Tm // TB),
        in_specs=[
            pl.BlockSpec((1, TB, 2, C), lambda n, t: (n, t, 0, 0)),
            pl.BlockSpec((1, TB, 2, C), lambda n, t: (n, t, 0, 0)),
            pl.BlockSpec((1, TB, 2, C), lambda n, t: (n, t, 0, 0)),
        ],
        out_specs=pl.BlockSpec((1, 1, 8, 128), lambda n, t: (n, t, 0, 0)),
        out_shape=jax.ShapeDtypeStruct((N, Tm // TB, 8, 128), jnp.float32),
        compiler_params=pltpu.CompilerParams(
            dimension_semantics=("parallel", "arbitrary"),
        ),
        name="chamfer_steps",
    )(ct[:, :-1], vt[:, :-1], ct[:, 1:])

    total = jnp.sum(out[:, :, 0, 0])
    return total * (0.5 / (N * C * Tm))


# speedup vs baseline: 6.4486x; 1.2818x over previous
import jax
import jax.numpy as jnp
from jax.experimental import pallas as pl
from jax.experimental.pallas import tpu as pltpu

MAX_H_BOUND = 1080.0
MAX_W_BOUND = 1920.0

TB = 16

_DN = (((0,), (0,)), ((), ()))


def _bf16x3(x):
    hi = x.astype(jnp.bfloat16)
    r = x - hi.astype(jnp.float32)
    mid = r.astype(jnp.bfloat16)
    lo = (r - mid.astype(jnp.float32)).astype(jnp.bfloat16)
    return hi, mid, lo


def _lane_min_dense(x):
    n = x.shape[1]
    while n > 128:
        n //= 2
        x = jnp.minimum(x[:, :n], x[:, n:])
    return _sublane_min(jnp.transpose(x))


def _sublane_min(x):
    n = x.shape[0]
    while n > 1:
        n //= 2
        x = jnp.minimum(x[:n, :], x[n:, :])
    return x


def _chamfer_steps_kernel(p_ref, v_ref, q_ref, o_ref):
    C = p_ref.shape[3]
    racc = jnp.zeros((1, C), jnp.float32)
    cacc = jnp.zeros((1, C), jnp.float32)
    for k in range(TB):
        pc = p_ref[0, k]
        vv = v_ref[0, k]
        qc = q_ref[0, k]

        s = pc + vv
        sx = jnp.clip(s[0:1, :], 0.0, MAX_H_BOUND)
        sy = jnp.clip(s[1:2, :], 0.0, MAX_W_BOUND)
        qx = qc[0:1, :]
        qy = qc[1:2, :]

        p2 = sx * sx + sy * sy
        q2 = qx * qx + qy * qy

        ones = jnp.ones((1, C), jnp.bfloat16)
        p2h, p2m, p2l = _bf16x3(p2)
        q2h, q2m, q2l = _bf16x3(q2)

        lhs = jnp.concatenate(
            [
                (-2.0 * sx).astype(jnp.bfloat16),
                (-2.0 * sy).astype(jnp.bfloat16),
                p2h, p2m, p2l,
                ones, ones, ones,
            ],
            axis=0,
        )
        rhs = jnp.concatenate(
            [
                qc.astype(jnp.bfloat16),
                ones, ones, ones,
                q2h, q2m, q2l,
            ],
            axis=0,
        )
        d2 = jax.lax.dot_general(lhs, rhs, _DN, preferred_element_type=jnp.float32)

        rmin = _lane_min_dense(d2)
        cmin = _sublane_min(d2)

        racc = racc + jnp.sqrt(jnp.maximum(rmin, 0.0))
        cacc = cacc + jnp.sqrt(jnp.maximum(cmin, 0.0))

    partial = jnp.sum(racc) + jnp.sum(cacc)
    o_ref[...] = jnp.full(o_ref.shape, partial, dtype=jnp.float32)


def kernel(center_tensor, velocity_vector):
    N, T, C, _ = center_tensor.shape
    Tm = T - 1
    ct = center_tensor.transpose(0, 1, 3, 2)
    vt = velocity_vector.transpose(0, 1, 3, 2)

    out = pl.pallas_call(
        _chamfer_steps_kernel,
        grid=(N, Tm // TB),
        in_specs=[
            pl.BlockSpec((1, TB, 2, C), lambda n, t: (n, t, 0, 0)),
            pl.BlockSpec((1, TB, 2, C), lambda n, t: (n, t, 0, 0)),
            pl.BlockSpec((1, TB, 2, C), lambda n, t: (n, t, 0, 0)),
        ],
        out_specs=pl.BlockSpec((1, 1, 8, 128), lambda n, t: (n, t, 0, 0)),
        out_shape=jax.ShapeDtypeStruct((N, Tm // TB, 8, 128), jnp.float32),
        compiler_params=pltpu.CompilerParams(
            dimension_semantics=("parallel", "arbitrary"),
        ),
        name="chamfer_steps",
    )(ct[:, :-1], vt[:, :-1], ct[:, 1:])

    total = jnp.sum(out[:, :, 0, 0])
    return total * (0.5 / (N * C * Tm))
